# baseline (device time: 35368 ns/iter reference)
import jax
import jax.numpy as jnp
from jax import lax
from jax.experimental import pallas as pl
from jax.experimental.pallas import tpu as pltpu


def kernel(O, Wo):
    b, s, h, d = O.shape
    k = h * d
    n = Wo.shape[1]
    s_half = s // 2

    def body(o_ref, w_ref, out_ref, comm_ref, send_sem, recv_sem):
        my_x = lax.axis_index("x")
        my_y = lax.axis_index("y")
        nbr_x = 1 - my_x

        barrier = pltpu.get_barrier_semaphore()
        pl.semaphore_signal(
            barrier, inc=1,
            device_id=(nbr_x, my_y), device_id_type=pl.DeviceIdType.MESH,
        )
        pl.semaphore_wait(barrier, 1)

        w = w_ref[...].astype(jnp.bfloat16)

        def partial_for(s_start):
            o_blk = o_ref[:, pl.ds(s_start, s_half), :, :]
            o_blk = o_blk.reshape(b * s_half, k).astype(jnp.bfloat16)
            return lax.dot_general(
                o_blk, w, (((1,), (0,)), ((), ())),
                preferred_element_type=jnp.float32,
            )

        p_nbr = partial_for(nbr_x * s_half)
        comm_ref[0] = p_nbr.astype(jnp.bfloat16).reshape(b, s_half, n)
        rdma = pltpu.make_async_remote_copy(
            src_ref=comm_ref.at[0],
            dst_ref=comm_ref.at[1],
            send_sem=send_sem,
            recv_sem=recv_sem,
            device_id=(nbr_x, my_y),
            device_id_type=pl.DeviceIdType.MESH,
        )
        rdma.start()

        p_mine = partial_for(my_x * s_half).reshape(b, s_half, n)

        rdma.wait()
        out_ref[...] = p_mine + comm_ref[1][...].astype(jnp.float32)

    return pl.pallas_call(
        body,
        out_shape=jax.ShapeDtypeStruct((b, s_half, n), jnp.float32),
        in_specs=[
            pl.BlockSpec(memory_space=pltpu.VMEM),
            pl.BlockSpec(memory_space=pltpu.VMEM),
        ],
        out_specs=pl.BlockSpec(memory_space=pltpu.VMEM),
        scratch_shapes=[
            pltpu.VMEM((2, b, s_half, n), jnp.bfloat16),
            pltpu.SemaphoreType.DMA,
            pltpu.SemaphoreType.DMA,
        ],
        compiler_params=pltpu.CompilerParams(collective_id=0),
    )(O, Wo)


# device time: 34399 ns/iter; 1.0282x vs baseline; 1.0282x over previous
import jax
import jax.numpy as jnp
from jax import lax
from jax.experimental import pallas as pl
from jax.experimental.pallas import tpu as pltpu


def kernel(O, Wo):
    b, s, h, d = O.shape
    k = h * d
    n = Wo.shape[1]
    s_half = s // 2

    def body(o_ref, w_ref, out_ref, comm_ref, send_sem, recv_sem):
        my_x = lax.axis_index("x")
        my_y = lax.axis_index("y")
        nbr_x = 1 - my_x

        barrier = pltpu.get_barrier_semaphore()
        pl.semaphore_signal(
            barrier, inc=1,
            device_id=(nbr_x, my_y), device_id_type=pl.DeviceIdType.MESH,
        )
        pl.semaphore_wait(barrier, 1)

        w = w_ref[...].astype(jnp.bfloat16)

        def partial_for(bi, s_start):
            o_blk = o_ref[bi, pl.ds(s_start, s_half), :, :]
            o_blk = o_blk.reshape(s_half, k).astype(jnp.bfloat16)
            return lax.dot_general(
                o_blk, w, (((1,), (0,)), ((), ())),
                preferred_element_type=jnp.float32,
            )

        rdmas = []
        for bi in range(b):
            comm_ref[0, bi] = partial_for(bi, nbr_x * s_half).astype(jnp.bfloat16)
            r = pltpu.make_async_remote_copy(
                src_ref=comm_ref.at[0, bi],
                dst_ref=comm_ref.at[1, bi],
                send_sem=send_sem.at[bi],
                recv_sem=recv_sem.at[bi],
                device_id=(nbr_x, my_y),
                device_id_type=pl.DeviceIdType.MESH,
            )
            r.start()
            rdmas.append(r)

        for bi in range(b):
            p_mine = partial_for(bi, my_x * s_half)
            rdmas[bi].wait()
            out_ref[bi] = p_mine + comm_ref[1, bi].astype(jnp.float32)

    return pl.pallas_call(
        body,
        out_shape=jax.ShapeDtypeStruct((b, s_half, n), jnp.float32),
        in_specs=[
            pl.BlockSpec(memory_space=pltpu.VMEM),
            pl.BlockSpec(memory_space=pltpu.VMEM),
        ],
        out_specs=pl.BlockSpec(memory_space=pltpu.VMEM),
        scratch_shapes=[
            pltpu.VMEM((2, b, s_half, n), jnp.bfloat16),
            pltpu.SemaphoreType.DMA((b,)),
            pltpu.SemaphoreType.DMA((b,)),
        ],
        compiler_params=pltpu.CompilerParams(collective_id=0),
    )(O, Wo)
